# baseline (speedup 1.0000x reference)
import functools

import jax
import jax.numpy as jnp
from jax import lax
from jax.experimental import pallas as pl
from jax.experimental.pallas import tpu as pltpu
from jax.experimental.pallas import tpu_sc as plsc

VOCAB = 30522
D = 768
B = 4096
L = 200

LANES = 16
NC = 2
NS = 16
NW = NC * NS
RB = B // NW
GROUPS = RB // LANES

VBLK = 5120
NVBLK = -(-VOCAB // VBLK)
VPAD = NVBLK * VBLK


def _matvec_body(emb_ref, wt_ref, b_ref, out_ref):
    x = emb_ref[...]
    w = wt_ref[...]
    out_ref[...] = jnp.sum(x * w, axis=1) + b_ref[0]


def _matvec(emb, wt, b):
    return pl.pallas_call(
        _matvec_body,
        grid=(NVBLK,),
        in_specs=[
            pl.BlockSpec((VBLK, D), lambda i: (i, 0)),
            pl.BlockSpec((1, D), lambda i: (0, 0)),
            pl.BlockSpec(memory_space=pltpu.SMEM),
        ],
        out_specs=pl.BlockSpec((VBLK,), lambda i: (i,)),
        out_shape=jax.ShapeDtypeStruct((VPAD,), jnp.float32),
    )(emb, wt, b)


@functools.partial(
    pl.kernel,
    out_type=jax.ShapeDtypeStruct((B,), jnp.float32),
    mesh=plsc.VectorSubcoreMesh(core_axis_name="c", subcore_axis_name="s"),
    compiler_params=pltpu.CompilerParams(
        needs_layout_passes=False,
        disable_bounds_checks=True,
        disable_semaphore_checks=True,
    ),
    scratch_types=[
        pltpu.VMEM((L, RB), jnp.int32),
        pltpu.VMEM((VPAD,), jnp.float32),
        pltpu.VMEM((RB,), jnp.float32),
        pltpu.SemaphoreType.DMA,
        pltpu.SemaphoreType.DMA,
    ],
)
def _sc_gather_mean(ids_hbm, v_hbm, out_hbm, ids_v, v_v, out_v, sem_v, sem_i):
    wid = lax.axis_index("s") * NC + lax.axis_index("c")
    cp_v = pltpu.async_copy(v_hbm, v_v, sem_v)
    cp_i = pltpu.async_copy(ids_hbm.at[:, pl.ds(wid * RB, RB)], ids_v, sem_i)
    cp_v.wait()
    cp_i.wait()
    zeros = jnp.zeros((LANES,), jnp.float32)

    def body(l, accs):
        new = []
        for g in range(GROUPS):
            ids16 = ids_v[l, pl.ds(g * LANES, LANES)]
            new.append(accs[g] + plsc.load_gather(v_v, [ids16]))
        return tuple(new)

    accs = lax.fori_loop(0, L, body, (zeros,) * GROUPS)
    inv_l = jnp.float32(1.0 / L)
    for g in range(GROUPS):
        out_v[pl.ds(g * LANES, LANES)] = accs[g] * inv_l
    pltpu.sync_copy(out_v, out_hbm.at[pl.ds(wid * RB, RB)])


def kernel(input_ids, emb_table, W, b):
    wt = W.reshape(1, D).astype(jnp.float32)
    v = _matvec(emb_table, wt, b.astype(jnp.float32))
    ids_t = input_ids.astype(jnp.int32).T
    out = _sc_gather_mean(ids_t, v)
    return out.reshape(B, 1)


# speedup vs baseline: 1.0039x; 1.0039x over previous
import functools

import jax
import jax.numpy as jnp
from jax import lax
from jax.experimental import pallas as pl
from jax.experimental.pallas import tpu as pltpu
from jax.experimental.pallas import tpu_sc as plsc

VOCAB = 30522
D = 768
B = 4096
L = 200

LANES = 16
NC = 2
NS = 16
NW = NC * NS
RB = B // NW
GROUPS = RB // LANES

VBLK = 4096
NVBLK = -(-VOCAB // VBLK)
VPAD = NVBLK * VBLK


def _matvec_body(emb_ref, wt_ref, b_ref, out_ref):
    x = emb_ref[...]
    w = wt_ref[...]
    out_ref[...] = jnp.sum(x * w, axis=1) + b_ref[0]


def _matvec(emb, wt, b):
    return pl.pallas_call(
        _matvec_body,
        grid=(NVBLK,),
        in_specs=[
            pl.BlockSpec((VBLK, D), lambda i: (i, 0)),
            pl.BlockSpec((1, D), lambda i: (0, 0)),
            pl.BlockSpec(memory_space=pltpu.SMEM),
        ],
        out_specs=pl.BlockSpec((VBLK,), lambda i: (i,)),
        out_shape=jax.ShapeDtypeStruct((VPAD,), jnp.float32),
    )(emb, wt, b)


@functools.partial(
    pl.kernel,
    out_type=jax.ShapeDtypeStruct((B,), jnp.float32),
    mesh=plsc.VectorSubcoreMesh(core_axis_name="c", subcore_axis_name="s"),
    compiler_params=pltpu.CompilerParams(
        needs_layout_passes=False,
        disable_bounds_checks=True,
        disable_semaphore_checks=True,
    ),
    scratch_types=[
        pltpu.VMEM((L, RB), jnp.int32),
        pltpu.VMEM((VPAD,), jnp.float32),
        pltpu.VMEM((RB,), jnp.float32),
        pltpu.SemaphoreType.DMA,
        pltpu.SemaphoreType.DMA,
    ],
)
def _sc_gather_mean(ids_hbm, v_hbm, out_hbm, ids_v, v_v, out_v, sem_v, sem_i):
    wid = lax.axis_index("s") * NC + lax.axis_index("c")
    cp_v = pltpu.async_copy(v_hbm, v_v, sem_v)
    cp_i = pltpu.async_copy(ids_hbm.at[:, pl.ds(wid * RB, RB)], ids_v, sem_i)
    cp_v.wait()
    cp_i.wait()
    zeros = jnp.zeros((LANES,), jnp.float32)

    def body(l, accs):
        new = []
        for g in range(GROUPS):
            ids16 = ids_v[l, pl.ds(g * LANES, LANES)]
            new.append(accs[g] + plsc.load_gather(v_v, [ids16]))
        return tuple(new)

    accs = lax.fori_loop(0, L, body, (zeros,) * GROUPS)
    inv_l = jnp.float32(1.0 / L)
    for g in range(GROUPS):
        out_v[pl.ds(g * LANES, LANES)] = accs[g] * inv_l
    pltpu.sync_copy(out_v, out_hbm.at[pl.ds(wid * RB, RB)])


def kernel(input_ids, emb_table, W, b):
    wt = W.reshape(1, D).astype(jnp.float32)
    v = _matvec(emb_table, wt, b.astype(jnp.float32))
    ids_t = input_ids.astype(jnp.int32).T
    out = _sc_gather_mean(ids_t, v)
    return out.reshape(B, 1)


# speedup vs baseline: 1.0084x; 1.0045x over previous
import functools

import jax
import jax.numpy as jnp
from jax import lax
from jax.experimental import pallas as pl
from jax.experimental.pallas import tpu as pltpu
from jax.experimental.pallas import tpu_sc as plsc

VOCAB = 30522
D = 768
B = 4096
L = 200

LANES = 16
NC = 2
NS = 16
NW = NC * NS
RB = B // NW
GROUPS = RB // LANES

VBLK = 3072
NVBLK = -(-VOCAB // VBLK)
VPAD = NVBLK * VBLK


def _matvec_body(emb_ref, wt_ref, b_ref, out_ref):
    x = emb_ref[...]
    w = wt_ref[...]
    out_ref[...] = jnp.sum(x * w, axis=1) + b_ref[0]


def _matvec(emb, wt, b):
    return pl.pallas_call(
        _matvec_body,
        grid=(NVBLK,),
        in_specs=[
            pl.BlockSpec((VBLK, D), lambda i: (i, 0)),
            pl.BlockSpec((1, D), lambda i: (0, 0)),
            pl.BlockSpec(memory_space=pltpu.SMEM),
        ],
        out_specs=pl.BlockSpec((VBLK,), lambda i: (i,)),
        out_shape=jax.ShapeDtypeStruct((VPAD,), jnp.float32),
    )(emb, wt, b)


@functools.partial(
    pl.kernel,
    out_type=jax.ShapeDtypeStruct((B,), jnp.float32),
    mesh=plsc.VectorSubcoreMesh(core_axis_name="c", subcore_axis_name="s"),
    compiler_params=pltpu.CompilerParams(
        needs_layout_passes=False,
        disable_bounds_checks=True,
        disable_semaphore_checks=True,
    ),
    scratch_types=[
        pltpu.VMEM((L, RB), jnp.int32),
        pltpu.VMEM((VPAD,), jnp.float32),
        pltpu.VMEM((RB,), jnp.float32),
        pltpu.SemaphoreType.DMA,
        pltpu.SemaphoreType.DMA,
    ],
)
def _sc_gather_mean(ids_hbm, v_hbm, out_hbm, ids_v, v_v, out_v, sem_v, sem_i):
    wid = lax.axis_index("s") * NC + lax.axis_index("c")
    cp_v = pltpu.async_copy(v_hbm, v_v, sem_v)
    cp_i = pltpu.async_copy(ids_hbm.at[:, pl.ds(wid * RB, RB)], ids_v, sem_i)
    cp_v.wait()
    cp_i.wait()
    zeros = jnp.zeros((LANES,), jnp.float32)

    @plsc.parallel_loop(0, L, unroll=2, carry=(zeros,) * GROUPS)
    def accs(l, accs):
        new = []
        for g in range(GROUPS):
            ids16 = ids_v[l, pl.ds(g * LANES, LANES)]
            new.append(accs[g] + plsc.load_gather(v_v, [ids16]))
        return tuple(new)
    inv_l = jnp.float32(1.0 / L)
    for g in range(GROUPS):
        out_v[pl.ds(g * LANES, LANES)] = accs[g] * inv_l
    pltpu.sync_copy(out_v, out_hbm.at[pl.ds(wid * RB, RB)])


def kernel(input_ids, emb_table, W, b):
    wt = W.reshape(1, D).astype(jnp.float32)
    v = _matvec(emb_table, wt, b.astype(jnp.float32))
    ids_t = input_ids.astype(jnp.int32).T
    out = _sc_gather_mean(ids_t, v)
    return out.reshape(B, 1)


# speedup vs baseline: 1.0112x; 1.0028x over previous
import functools

import jax
import jax.numpy as jnp
from jax import lax
from jax.experimental import pallas as pl
from jax.experimental.pallas import tpu as pltpu
from jax.experimental.pallas import tpu_sc as plsc

VOCAB = 30522
D = 768
B = 4096
L = 200

LANES = 16
NC = 2
NS = 16
NW = NC * NS
RB = B // NW
GROUPS = RB // LANES

VBLK = 3072
NVBLK = -(-VOCAB // VBLK)
VPAD = NVBLK * VBLK


def _matvec_body(emb_ref, wt_ref, b_ref, out_ref):
    x = emb_ref[...]
    w = wt_ref[...]
    out_ref[...] = jnp.sum(x * w, axis=1) + b_ref[0]


def _matvec(emb, wt, b):
    return pl.pallas_call(
        _matvec_body,
        grid=(NVBLK,),
        in_specs=[
            pl.BlockSpec((VBLK, D), lambda i: (i, 0)),
            pl.BlockSpec((1, D), lambda i: (0, 0)),
            pl.BlockSpec(memory_space=pltpu.SMEM),
        ],
        out_specs=pl.BlockSpec((VBLK,), lambda i: (i,)),
        out_shape=jax.ShapeDtypeStruct((VPAD,), jnp.float32),
    )(emb, wt, b)


@functools.partial(
    pl.kernel,
    out_type=jax.ShapeDtypeStruct((B,), jnp.float32),
    mesh=plsc.VectorSubcoreMesh(core_axis_name="c", subcore_axis_name="s"),
    compiler_params=pltpu.CompilerParams(needs_layout_passes=False),
    scratch_types=[
        pltpu.VMEM((L, RB), jnp.int32),
        pltpu.VMEM((VPAD,), jnp.float32),
        pltpu.VMEM((RB,), jnp.float32),
        pltpu.SemaphoreType.DMA,
        pltpu.SemaphoreType.DMA,
    ],
)
def _sc_gather_mean(ids_hbm, v_hbm, out_hbm, ids_v, v_v, out_v, sem_v, sem_i):
    wid = lax.axis_index("s") * NC + lax.axis_index("c")
    cp_v = pltpu.async_copy(v_hbm, v_v, sem_v)
    cp_i = pltpu.async_copy(ids_hbm.at[:, pl.ds(wid * RB, RB)], ids_v, sem_i)
    cp_v.wait()
    cp_i.wait()
    zeros = jnp.zeros((LANES,), jnp.float32)

    @plsc.parallel_loop(0, L, unroll=2, carry=(zeros,) * GROUPS)
    def accs(l, accs):
        new = []
        for g in range(GROUPS):
            ids16 = ids_v[l, pl.ds(g * LANES, LANES)]
            new.append(accs[g] + plsc.load_gather(v_v, [ids16]))
        return tuple(new)
    inv_l = jnp.float32(1.0 / L)
    for g in range(GROUPS):
        out_v[pl.ds(g * LANES, LANES)] = accs[g] * inv_l
    pltpu.sync_copy(out_v, out_hbm.at[pl.ds(wid * RB, RB)])


def kernel(input_ids, emb_table, W, b):
    wt = W.reshape(1, D).astype(jnp.float32)
    v = _matvec(emb_table, wt, b.astype(jnp.float32))
    ids_t = input_ids.astype(jnp.int32).T
    out = _sc_gather_mean(ids_t, v)
    return out.reshape(B, 1)
